# baseline (speedup 1.0000x reference)
import functools

import jax
import jax.numpy as jnp
from jax import lax
from jax.experimental import pallas as pl
from jax.experimental.pallas import tpu as pltpu
from jax.experimental.pallas import tpu_sc as plsc

_B, _T, _F = 16, 2048, 55
_EMB = 32
_OUT = 128
_NTOK = _B * _T
_NLOOK = 10
_GROUP = 128
_NW = 32
_TPW = _NTOK // _NW
_NH = 4
_HTOK = _TPW // _NH
_BPH = _HTOK // _GROUP
_GPW = _NH * _NLOOK * _BPH
_NG = _NW * _GPW

_IDX_COLS = (2, 3, 4, 5, 6, 27, 31, 33, 41, 49)
_TAB_OF_L = (0, 0, 0, 0, 0, 1, 2, 2, 2, 3)
_PASS_COLS = ((0, 1) + tuple(range(7, 27)) + (28, 29, 30) + (32,)
              + tuple(range(34, 41)) + tuple(range(42, 49))
              + tuple(range(50, 55)))

_TOK_BLK = 512
_GRID = _NTOK // _TOK_BLK
_BLK_PER_BATCH = _T // _TOK_BLK


def _premix_body(t_ref, w_ref, m_ref):
    m_ref[...] = jnp.dot(t_ref[...], w_ref[0],
                         preferred_element_type=jnp.float32)


def _premix(t10, wemb10):
    return pl.pallas_call(
        _premix_body,
        grid=(_NLOOK,),
        in_specs=[
            pl.BlockSpec((1000, _EMB), lambda l: (l, 0)),
            pl.BlockSpec((1, _EMB, _OUT), lambda l: (l, 0, 0)),
        ],
        out_specs=pl.BlockSpec((1000, _OUT), lambda l: (l, 0)),
        out_shape=jax.ShapeDtypeStruct((_NLOOK * 1000, _OUT), jnp.float32),
    )(t10, wemb10)


def _sc_embsum(m_tab, idx2d, zpage):
    mesh = plsc.VectorSubcoreMesh(core_axis_name="c", subcore_axis_name="s")

    @functools.partial(
        pl.kernel,
        out_type=jax.ShapeDtypeStruct((_NTOK, _OUT), jnp.float32),
        mesh=mesh,
        scratch_types=[
            pltpu.VMEM((_GPW, _GROUP), jnp.int32),
            pltpu.VMEM((2 * _HTOK, _OUT), jnp.float32),
            pltpu.SemaphoreType.DMA,
            pltpu.SemaphoreType.DMA,
            pltpu.SemaphoreType.DMA,
            pltpu.SemaphoreType.DMA,
        ],
    )
    def sc_kernel(m_hbm, idx_hbm, zero_hbm, out_hbm, idx_v, acc_v, gsem,
                  csem0, csem1, bsem):
        sid = lax.axis_index("s")
        wid = sid * 2 + lax.axis_index("c")
        pltpu.sync_copy(idx_hbm.at[pl.ds(wid * _GPW, _GPW)], idx_v)
        cout = [None, None]
        for h in range(_NH):
            a = h % 2
            tok0 = wid * _TPW + h * _HTOK
            if cout[a] is not None:
                cout[a].wait()
            pltpu.async_copy(
                zero_hbm, acc_v.at[pl.ds(a * _HTOK, _HTOK)], bsem).wait()
            adds = []
            for bq in range(_BPH):
                for l in range(_NLOOK):
                    g = (h * _NLOOK + l) * _BPH + bq
                    adds.append(pltpu.async_copy(
                        m_hbm.at[idx_v.at[g]],
                        acc_v.at[pl.ds(a * _HTOK + bq * _GROUP, _GROUP)],
                        gsem, add=True))
            for cp in adds:
                cp.wait()
            cout[a] = pltpu.async_copy(
                acc_v.at[pl.ds(a * _HTOK, _HTOK)],
                out_hbm.at[pl.ds(tok0, _HTOK)], csem0 if a == 0 else csem1)
        for cp in cout:
            if cp is not None:
                cp.wait()

    return sc_kernel(m_tab, idx2d, zpage)


def _spk_body(sids_ref, spk_hbm, out_ref, sem):
    cps = []
    for bt in range(_B):
        cp = pltpu.make_async_copy(
            spk_hbm.at[pl.ds(sids_ref[bt], 1)],
            out_ref.at[pl.ds(bt, 1)], sem)
        cp.start()
        cps.append(cp)
    for cp in cps:
        cp.wait()


def _spk_fetch(sids, spk_table):
    return pl.pallas_call(
        _spk_body,
        grid_spec=pltpu.PrefetchScalarGridSpec(
            num_scalar_prefetch=1,
            in_specs=[pl.BlockSpec(memory_space=pl.ANY)],
            out_specs=pl.BlockSpec(memory_space=pltpu.MemorySpace.VMEM),
            scratch_shapes=[pltpu.SemaphoreType.DMA],
        ),
        out_shape=jax.ShapeDtypeStruct((_B, 64), jnp.float32),
    )(sids, spk_table)


def _mix_body(emb_ref, utt_ref, spk_ref, wp55_ref, wspkT_ref, b_ref, out_ref):
    acc = (emb_ref[...]
           + jnp.dot(utt_ref[...], wp55_ref[...],
                     preferred_element_type=jnp.float32))
    bidx = pl.program_id(0) // _BLK_PER_BATCH
    spk_row = spk_ref[pl.ds(bidx, 1), :]
    smix = jnp.dot(spk_row, wspkT_ref[...],
                   preferred_element_type=jnp.float32)
    out_ref[...] = acc + smix + b_ref[...]


def _tc_mix(embsum, utt2d, spk_rows, Wp55, WspkT, b2d):
    return pl.pallas_call(
        _mix_body,
        grid=(_GRID,),
        in_specs=[
            pl.BlockSpec((_TOK_BLK, _OUT), lambda i: (i, 0)),
            pl.BlockSpec((_TOK_BLK, _F), lambda i: (i, 0)),
            pl.BlockSpec((_B, 64), lambda i: (0, 0)),
            pl.BlockSpec((_F, _OUT), lambda i: (0, 0)),
            pl.BlockSpec((64, _OUT), lambda i: (0, 0)),
            pl.BlockSpec((1, _OUT), lambda i: (0, 0)),
        ],
        out_specs=pl.BlockSpec((_TOK_BLK, _OUT), lambda i: (i, 0)),
        out_shape=jax.ShapeDtypeStruct((_NTOK, _OUT), jnp.float32),
    )(embsum, utt2d, spk_rows, Wp55, WspkT, b2d)


def kernel(utt_conds, speakers_ids, speaker_table, phon_table, vowel_table,
           gpos_table, tobi_table, W, b):
    utt2d = utt_conds.reshape(_NTOK, _F)

    idx = (utt2d[:, jnp.array(_IDX_COLS)].astype(jnp.int32)
           + (jnp.arange(_NLOOK, dtype=jnp.int32) * 1000)[None, :])
    idx2d = (idx.reshape(_NW, _NH, _BPH, _GROUP, _NLOOK)
             .transpose(0, 1, 4, 2, 3).reshape(_NG, _GROUP))

    t10 = jnp.concatenate(
        [phon_table] * 5 + [vowel_table] + [gpos_table] * 3 + [tobi_table],
        axis=0)
    wemb10 = W[:, 64:64 + _NLOOK * _EMB].T.reshape(_NLOOK, _EMB, _OUT)
    m_tab = _premix(t10, wemb10)

    spk_rows = _spk_fetch(speakers_ids.astype(jnp.int32), speaker_table)

    WspkT = W[:, :64].T
    Wp55 = (jnp.zeros((_F, _OUT), W.dtype)
            .at[jnp.array(_PASS_COLS)].set(W[:, 64 + _NLOOK * _EMB:].T))

    zpage = jnp.zeros((_HTOK, _OUT), jnp.float32)
    embsum = _sc_embsum(m_tab, idx2d, zpage)
    out2d = _tc_mix(embsum, utt2d, spk_rows, Wp55, WspkT, b.reshape(1, _OUT))
    return out2d.reshape(_B, _T, _OUT)


# speedup vs baseline: 1.0365x; 1.0365x over previous
import functools

import jax
import jax.numpy as jnp
from jax import lax
from jax.experimental import pallas as pl
from jax.experimental.pallas import tpu as pltpu
from jax.experimental.pallas import tpu_sc as plsc

_B, _T, _F = 16, 2048, 55
_EMB = 32
_OUT = 128
_NTOK = _B * _T
_NLOOK = 10
_GROUP = 128
_NW = 32
_TPW = _NTOK // _NW
_NH = 4
_HTOK = _TPW // _NH
_BPH = _HTOK // _GROUP
_GPW = _NH * _NLOOK * _BPH
_NG = _NW * _GPW

_IDX_COLS = (2, 3, 4, 5, 6, 27, 31, 33, 41, 49)
_TAB_OF_L = (0, 0, 0, 0, 0, 1, 2, 2, 2, 3)
_PASS_COLS = ((0, 1) + tuple(range(7, 27)) + (28, 29, 30) + (32,)
              + tuple(range(34, 41)) + tuple(range(42, 49))
              + tuple(range(50, 55)))

_TOK_BLK = 512
_GRID = _NTOK // _TOK_BLK
_BLK_PER_BATCH = _T // _TOK_BLK


def _premix_body(t_ref, w_ref, m_ref):
    m_ref[...] = jnp.dot(t_ref[...], w_ref[0],
                         preferred_element_type=jnp.float32)


def _premix(t10, wemb10):
    return pl.pallas_call(
        _premix_body,
        grid=(_NLOOK,),
        in_specs=[
            pl.BlockSpec((1000, _EMB), lambda l: (l, 0)),
            pl.BlockSpec((1, _EMB, _OUT), lambda l: (l, 0, 0)),
        ],
        out_specs=pl.BlockSpec((1000, _OUT), lambda l: (l, 0)),
        out_shape=jax.ShapeDtypeStruct((_NLOOK * 1000, _OUT), jnp.float32),
    )(t10, wemb10)


def _sc_embsum(m_tab, idx2d, zpage):
    mesh = plsc.VectorSubcoreMesh(core_axis_name="c", subcore_axis_name="s")

    @functools.partial(
        pl.kernel,
        out_type=jax.ShapeDtypeStruct((_NTOK, _OUT), jnp.float32),
        mesh=mesh,
        scratch_types=[
            pltpu.VMEM((_GPW, _GROUP), jnp.int32),
            pltpu.VMEM((2 * _HTOK, _OUT), jnp.float32),
            pltpu.SemaphoreType.DMA,
            pltpu.SemaphoreType.DMA,
            pltpu.SemaphoreType.DMA,
            pltpu.SemaphoreType.DMA,
        ],
    )
    def sc_kernel(m_hbm, idx_hbm, zero_hbm, out_hbm, idx_v, acc_v, gsem,
                  csem0, csem1, bsem):
        sid = lax.axis_index("s")
        wid = sid * 2 + lax.axis_index("c")
        pltpu.sync_copy(idx_hbm.at[pl.ds(wid * _GPW, _GPW)], idx_v)
        cout = [None, None]
        for h in range(_NH):
            a = h % 2
            tok0 = wid * _TPW + h * _HTOK
            if cout[a] is not None:
                cout[a].wait()
            pltpu.async_copy(
                zero_hbm.at[pl.ds(wid * _HTOK, _HTOK)],
                acc_v.at[pl.ds(a * _HTOK, _HTOK)], bsem).wait()
            adds = []
            for bq in range(_BPH):
                for l in range(_NLOOK):
                    g = (h * _NLOOK + l) * _BPH + bq
                    adds.append(pltpu.async_copy(
                        m_hbm.at[idx_v.at[g]],
                        acc_v.at[pl.ds(a * _HTOK + bq * _GROUP, _GROUP)],
                        gsem, add=True))
            for cp in adds:
                cp.wait()
            cout[a] = pltpu.async_copy(
                acc_v.at[pl.ds(a * _HTOK, _HTOK)],
                out_hbm.at[pl.ds(tok0, _HTOK)], csem0 if a == 0 else csem1)
        for cp in cout:
            if cp is not None:
                cp.wait()

    return sc_kernel(m_tab, idx2d, zpage)


def _spk_body(sids_ref, spk_hbm, out_ref, sem):
    cps = []
    for bt in range(_B):
        cp = pltpu.make_async_copy(
            spk_hbm.at[pl.ds(sids_ref[bt], 1)],
            out_ref.at[pl.ds(bt, 1)], sem)
        cp.start()
        cps.append(cp)
    for cp in cps:
        cp.wait()


def _spk_fetch(sids, spk_table):
    return pl.pallas_call(
        _spk_body,
        grid_spec=pltpu.PrefetchScalarGridSpec(
            num_scalar_prefetch=1,
            in_specs=[pl.BlockSpec(memory_space=pl.ANY)],
            out_specs=pl.BlockSpec(memory_space=pltpu.MemorySpace.VMEM),
            scratch_shapes=[pltpu.SemaphoreType.DMA],
        ),
        out_shape=jax.ShapeDtypeStruct((_B, 64), jnp.float32),
    )(sids, spk_table)


def _mix_body(emb_ref, utt_ref, spk_ref, wp55_ref, wspkT_ref, b_ref, out_ref):
    acc = (emb_ref[...]
           + jnp.dot(utt_ref[...], wp55_ref[...],
                     preferred_element_type=jnp.float32))
    bidx = pl.program_id(0) // _BLK_PER_BATCH
    spk_row = spk_ref[pl.ds(bidx, 1), :]
    smix = jnp.dot(spk_row, wspkT_ref[...],
                   preferred_element_type=jnp.float32)
    out_ref[...] = acc + smix + b_ref[...]


def _tc_mix(embsum, utt2d, spk_rows, Wp55, WspkT, b2d):
    return pl.pallas_call(
        _mix_body,
        grid=(_GRID,),
        in_specs=[
            pl.BlockSpec((_TOK_BLK, _OUT), lambda i: (i, 0)),
            pl.BlockSpec((_TOK_BLK, _F), lambda i: (i, 0)),
            pl.BlockSpec((_B, 64), lambda i: (0, 0)),
            pl.BlockSpec((_F, _OUT), lambda i: (0, 0)),
            pl.BlockSpec((64, _OUT), lambda i: (0, 0)),
            pl.BlockSpec((1, _OUT), lambda i: (0, 0)),
        ],
        out_specs=pl.BlockSpec((_TOK_BLK, _OUT), lambda i: (i, 0)),
        out_shape=jax.ShapeDtypeStruct((_NTOK, _OUT), jnp.float32),
    )(embsum, utt2d, spk_rows, Wp55, WspkT, b2d)


def kernel(utt_conds, speakers_ids, speaker_table, phon_table, vowel_table,
           gpos_table, tobi_table, W, b):
    utt2d = utt_conds.reshape(_NTOK, _F)

    idx = (utt2d[:, jnp.array(_IDX_COLS)].astype(jnp.int32)
           + (jnp.arange(_NLOOK, dtype=jnp.int32) * 1000)[None, :])
    idx2d = (idx.reshape(_NW, _NH, _BPH, _GROUP, _NLOOK)
             .transpose(0, 1, 4, 2, 3).reshape(_NG, _GROUP))

    t10 = jnp.concatenate(
        [phon_table] * 5 + [vowel_table] + [gpos_table] * 3 + [tobi_table],
        axis=0)
    wemb10 = W[:, 64:64 + _NLOOK * _EMB].T.reshape(_NLOOK, _EMB, _OUT)
    m_tab = _premix(t10, wemb10)

    spk_rows = _spk_fetch(speakers_ids.astype(jnp.int32), speaker_table)

    WspkT = W[:, :64].T
    Wp55 = (jnp.zeros((_F, _OUT), W.dtype)
            .at[jnp.array(_PASS_COLS)].set(W[:, 64 + _NLOOK * _EMB:].T))

    zpage = jnp.zeros((_NW * _HTOK, _OUT), jnp.float32)
    embsum = _sc_embsum(m_tab, idx2d, zpage)
    out2d = _tc_mix(embsum, utt2d, spk_rows, Wp55, WspkT, b.reshape(1, _OUT))
    return out2d.reshape(_B, _T, _OUT)


# speedup vs baseline: 1.0751x; 1.0372x over previous
import functools

import jax
import jax.numpy as jnp
from jax import lax
from jax.experimental import pallas as pl
from jax.experimental.pallas import tpu as pltpu
from jax.experimental.pallas import tpu_sc as plsc

_B, _T, _F = 16, 2048, 55
_EMB = 32
_OUT = 128
_NTOK = _B * _T
_NLOOK = 10
_GROUP = 128
_NW = 32
_TPW = _NTOK // _NW
_NH = 4
_HTOK = _TPW // _NH
_BPH = _HTOK // _GROUP
_GPW = _NH * _NLOOK * _BPH
_NG = _NW * _GPW

_IDX_COLS = (2, 3, 4, 5, 6, 27, 31, 33, 41, 49)
_TAB_OF_L = (0, 0, 0, 0, 0, 1, 2, 2, 2, 3)
_PASS_COLS = ((0, 1) + tuple(range(7, 27)) + (28, 29, 30) + (32,)
              + tuple(range(34, 41)) + tuple(range(42, 49))
              + tuple(range(50, 55)))

_TOK_BLK = 512
_GRID = _NTOK // _TOK_BLK
_BLK_PER_BATCH = _T // _TOK_BLK


def _premix_body(t_ref, w_ref, m_ref):
    m_ref[...] = jnp.dot(t_ref[...], w_ref[0],
                         preferred_element_type=jnp.float32)


def _premix(t10, wemb10):
    return pl.pallas_call(
        _premix_body,
        grid=(_NLOOK,),
        in_specs=[
            pl.BlockSpec((1000, _EMB), lambda l: (l, 0)),
            pl.BlockSpec((1, _EMB, _OUT), lambda l: (l, 0, 0)),
        ],
        out_specs=pl.BlockSpec((1000, _OUT), lambda l: (l, 0)),
        out_shape=jax.ShapeDtypeStruct((_NLOOK * 1000, _OUT), jnp.float32),
    )(t10, wemb10)


def _sc_embsum(m_tab, idx2d):
    mesh = plsc.VectorSubcoreMesh(core_axis_name="c", subcore_axis_name="s")

    @functools.partial(
        pl.kernel,
        out_type=jax.ShapeDtypeStruct((_NTOK, _OUT), jnp.float32),
        mesh=mesh,
        scratch_types=[
            pltpu.VMEM((_GPW, _GROUP), jnp.int32),
            pltpu.VMEM((2 * _HTOK, _OUT), jnp.float32),
            pltpu.SemaphoreType.DMA,
            pltpu.SemaphoreType.DMA,
            pltpu.SemaphoreType.DMA,
        ],
    )
    def sc_kernel(m_hbm, idx_hbm, out_hbm, idx_v, acc_v, gsem, csem0, csem1):
        sid = lax.axis_index("s")
        wid = sid * 2 + lax.axis_index("c")
        pltpu.sync_copy(idx_hbm.at[pl.ds(wid * _GPW, _GPW)], idx_v)
        cout = [None, None]
        for h in range(_NH):
            a = h % 2
            if cout[a] is not None:
                cout[a].wait()
            g0 = []
            for bq in range(_BPH):
                g = (h * _NLOOK) * _BPH + bq
                g0.append(pltpu.async_copy(
                    m_hbm.at[idx_v.at[g]],
                    acc_v.at[pl.ds(a * _HTOK + bq * _GROUP, _GROUP)], gsem))
            adds = []
            for bq in range(_BPH):
                g0[bq].wait()
                for l in range(1, _NLOOK):
                    g = (h * _NLOOK + l) * _BPH + bq
                    adds.append(pltpu.async_copy(
                        m_hbm.at[idx_v.at[g]],
                        acc_v.at[pl.ds(a * _HTOK + bq * _GROUP, _GROUP)],
                        gsem, add=True))
            for cp in adds:
                cp.wait()
            cout[a] = pltpu.async_copy(
                acc_v.at[pl.ds(a * _HTOK, _HTOK)],
                out_hbm.at[pl.ds(wid * _TPW + h * _HTOK, _HTOK)],
                csem0 if a == 0 else csem1)
        for cp in cout:
            if cp is not None:
                cp.wait()

    return sc_kernel(m_tab, idx2d)


def _spk_body(sids_ref, spk_hbm, out_ref, sem):
    cps = []
    for bt in range(_B):
        cp = pltpu.make_async_copy(
            spk_hbm.at[pl.ds(sids_ref[bt], 1)],
            out_ref.at[pl.ds(bt, 1)], sem)
        cp.start()
        cps.append(cp)
    for cp in cps:
        cp.wait()


def _spk_fetch(sids, spk_table):
    return pl.pallas_call(
        _spk_body,
        grid_spec=pltpu.PrefetchScalarGridSpec(
            num_scalar_prefetch=1,
            in_specs=[pl.BlockSpec(memory_space=pl.ANY)],
            out_specs=pl.BlockSpec(memory_space=pltpu.MemorySpace.VMEM),
            scratch_shapes=[pltpu.SemaphoreType.DMA],
        ),
        out_shape=jax.ShapeDtypeStruct((_B, 64), jnp.float32),
    )(sids, spk_table)


def _mix_body(emb_ref, utt_ref, spk_ref, wp55_ref, wspkT_ref, b_ref, out_ref):
    acc = (emb_ref[...].astype(jnp.float32)
           + jnp.dot(utt_ref[...], wp55_ref[...],
                     preferred_element_type=jnp.float32))
    bidx = pl.program_id(0) // _BLK_PER_BATCH
    spk_row = spk_ref[pl.ds(bidx, 1), :]
    smix = jnp.dot(spk_row, wspkT_ref[...],
                   preferred_element_type=jnp.float32)
    out_ref[...] = acc + smix + b_ref[...]


def _tc_mix(embsum, utt2d, spk_rows, Wp55, WspkT, b2d):
    return pl.pallas_call(
        _mix_body,
        grid=(_GRID,),
        in_specs=[
            pl.BlockSpec((_TOK_BLK, _OUT), lambda i: (i, 0)),
            pl.BlockSpec((_TOK_BLK, _F), lambda i: (i, 0)),
            pl.BlockSpec((_B, 64), lambda i: (0, 0)),
            pl.BlockSpec((_F, _OUT), lambda i: (0, 0)),
            pl.BlockSpec((64, _OUT), lambda i: (0, 0)),
            pl.BlockSpec((1, _OUT), lambda i: (0, 0)),
        ],
        out_specs=pl.BlockSpec((_TOK_BLK, _OUT), lambda i: (i, 0)),
        out_shape=jax.ShapeDtypeStruct((_NTOK, _OUT), jnp.float32),
    )(embsum, utt2d, spk_rows, Wp55, WspkT, b2d)


def kernel(utt_conds, speakers_ids, speaker_table, phon_table, vowel_table,
           gpos_table, tobi_table, W, b):
    utt2d = utt_conds.reshape(_NTOK, _F)

    idx = (utt2d[:, jnp.array(_IDX_COLS)].astype(jnp.int32)
           + (jnp.arange(_NLOOK, dtype=jnp.int32) * 1000)[None, :])
    idx2d = (idx.reshape(_NW, _NH, _BPH, _GROUP, _NLOOK)
             .transpose(0, 1, 4, 2, 3).reshape(_NG, _GROUP))

    t10 = jnp.concatenate(
        [phon_table] * 5 + [vowel_table] + [gpos_table] * 3 + [tobi_table],
        axis=0)
    wemb10 = W[:, 64:64 + _NLOOK * _EMB].T.reshape(_NLOOK, _EMB, _OUT)
    m_tab = _premix(t10, wemb10)

    embsum = _sc_embsum(m_tab, idx2d)
    spk_rows = _spk_fetch(speakers_ids.astype(jnp.int32), speaker_table)

    WspkT = W[:, :64].T
    Wp55 = (jnp.zeros((_F, _OUT), W.dtype)
            .at[jnp.array(_PASS_COLS)].set(W[:, 64 + _NLOOK * _EMB:].T))

    out2d = _tc_mix(embsum, utt2d, spk_rows, Wp55, WspkT, b.reshape(1, _OUT))
    return out2d.reshape(_B, _T, _OUT)
